# baseline (speedup 1.0000x reference)
import functools

import jax
import jax.numpy as jnp
from jax import lax
from jax.experimental import pallas as pl
from jax.experimental.pallas import tpu as pltpu
from jax.experimental.pallas import tpu_sc as plsc

F32 = jnp.float32
I32 = jnp.int32

_USER_NUM = 5000
_ITEM_NUM = 5000
NNODE = _USER_NUM + _ITEM_NUM
D = 128
B = 4096
NC, NS, LANES = 2, 16, 16
NW = NC * NS
CHUNK = 128
CH_E = 80
EPAD = NW * CH_E * CHUNK
N_ACC = 10240
ZROWS = N_ACC // NS
OROWS = N_ACC // NS
BGK = (3 * B) // (NW * CHUNK)

_mesh = plsc.VectorSubcoreMesh(
    core_axis_name="c", subcore_axis_name="s", num_cores=NC, num_subcores=NS)


@functools.partial(
    pl.kernel,
    out_type=jax.ShapeDtypeStruct((NC, N_ACC, D), F32),
    mesh=_mesh,
    scratch_types=[
        pltpu.VMEM((CH_E, CHUNK), I32),
        pltpu.VMEM((CHUNK, D), F32),
        pltpu.VMEM_SHARED((N_ACC, D), F32),
    ],
)
def _deg_kernel(col_hbm, out_hbm, col_v, ones_v, acc):
    cid = lax.axis_index("c")
    sid = lax.axis_index("s")
    wid = cid * NS + sid
    pltpu.sync_copy(col_hbm.at[wid], col_v)
    zero = jnp.zeros((LANES,), F32)

    def _zb(i, _):
        for k in range(D // LANES):
            ones_v[i, pl.ds(k * LANES, LANES)] = zero
        return 0

    lax.fori_loop(0, CHUNK, _zb, 0)
    base = sid * ZROWS
    for k in range(ZROWS // CHUNK):
        pltpu.sync_copy(ones_v, acc.at[pl.ds(base + k * CHUNK, CHUNK)])

    one = jnp.ones((LANES,), F32)

    def _ob(i, _):
        for k in range(D // LANES):
            ones_v[i, pl.ds(k * LANES, LANES)] = one
        return 0

    lax.fori_loop(0, CHUNK, _ob, 0)
    plsc.subcore_barrier()

    def _body(j, _):
        pltpu.sync_copy(ones_v, acc.at[col_v.at[j]], add=True)
        return 0

    lax.fori_loop(0, CH_E, _body, 0)
    plsc.subcore_barrier()
    ob = sid * OROWS
    pltpu.sync_copy(acc.at[pl.ds(ob, OROWS)], out_hbm.at[cid, pl.ds(ob, OROWS)])


GCHUNK = 64
GDEPTH = 4
GPH = 4
GCH = EPAD // (NW * GCHUNK)
GQ = GCH // GPH


@functools.partial(
    pl.kernel,
    out_type=jax.ShapeDtypeStruct((NC, N_ACC, D), F32),
    mesh=_mesh,
    scratch_types=[
        pltpu.VMEM((GQ, GCHUNK), I32),
        pltpu.VMEM((GQ, GCHUNK), I32),
    ] + [pltpu.VMEM((GCHUNK, D), F32) for _ in range(GDEPTH)]
      + [pltpu.SemaphoreType.DMA for _ in range(GDEPTH)]
      + [pltpu.VMEM_SHARED((N_ACC, D), F32)],
)
def _agg_kernel(h_hbm, row_hbm, col_hbm, out_hbm, row_v, col_v, *rest):
    bufs = rest[:GDEPTH]
    sems = rest[GDEPTH:2 * GDEPTH]
    acc = rest[2 * GDEPTH]
    cid = lax.axis_index("c")
    sid = lax.axis_index("s")
    wid = cid * NS + sid
    zero = jnp.zeros((LANES,), F32)

    def _zb(i, _):
        for k in range(D // LANES):
            bufs[0][i, pl.ds(k * LANES, LANES)] = zero
            bufs[1][i, pl.ds(k * LANES, LANES)] = zero
        return 0

    lax.fori_loop(0, GCHUNK, _zb, 0)
    base = sid * ZROWS
    for k in range(ZROWS // (2 * GCHUNK)):
        pltpu.sync_copy(bufs[0], acc.at[pl.ds(base + (2 * k) * GCHUNK, GCHUNK)])
        pltpu.sync_copy(bufs[1], acc.at[pl.ds(base + (2 * k + 1) * GCHUNK, GCHUNK)])
    plsc.subcore_barrier()

    def _start(j, b):
        pltpu.async_copy(h_hbm.at[row_v.at[j]], bufs[b], sems[b])

    def _wait(j, b):
        pltpu.make_async_copy(h_hbm.at[row_v.at[j]], bufs[b], sems[b]).wait()

    for ph in range(GPH):
        pltpu.sync_copy(row_hbm.at[wid, pl.ds(ph * GQ, GQ)], row_v)
        pltpu.sync_copy(col_hbm.at[wid, pl.ds(ph * GQ, GQ)], col_v)
        for b in range(GDEPTH):
            _start(b, b)

        def _body(g, _):
            j0 = g * GDEPTH
            for b in range(GDEPTH):
                _wait(j0 + b, b)
                pltpu.sync_copy(bufs[b], acc.at[col_v.at[j0 + b]], add=True)
            for b in range(GDEPTH):
                _start(j0 + GDEPTH + b, b)
            return 0

        lax.fori_loop(0, GQ // GDEPTH - 1, _body, 0)
        jl = GQ - GDEPTH
        for b in range(GDEPTH):
            _wait(jl + b, b)
            pltpu.sync_copy(bufs[b], acc.at[col_v.at[jl + b]], add=True)
    plsc.subcore_barrier()
    ob = sid * OROWS
    pltpu.sync_copy(acc.at[pl.ds(ob, OROWS)], out_hbm.at[cid, pl.ds(ob, OROWS)])


@functools.partial(
    pl.kernel,
    out_type=jax.ShapeDtypeStruct((3 * B, D), F32),
    mesh=_mesh,
    scratch_types=[
        pltpu.VMEM((BGK, CHUNK), I32),
        pltpu.VMEM((CHUNK, D), F32),
        pltpu.SemaphoreType.DMA,
    ],
)
def _bpr_gather(x_hbm, idx_hbm, out_hbm, idx_v, buf, sem):
    cid = lax.axis_index("c")
    sid = lax.axis_index("s")
    wid = cid * NS + sid
    pltpu.sync_copy(idx_hbm.at[wid], idx_v)
    for j in range(BGK):
        pltpu.async_copy(x_hbm.at[idx_v.at[j]], buf, sem).wait()
        pltpu.sync_copy(buf, out_hbm.at[pl.ds((wid * BGK + j) * CHUNK, CHUNK)])


BN_ROWS = 2000
GRID_N = NNODE // BN_ROWS


def _dis_block(degp_ref):
    deg = degp_ref[0, :, 0:1] + degp_ref[1, :, 0:1] + 1.0
    return lax.rsqrt(deg)


def _tc1_body(x_ref, w_ref, degp_ref, out_ref):
    dis = _dis_block(degp_ref)
    xw = jnp.dot(x_ref[...], w_ref[...], precision=lax.Precision.HIGHEST)
    out_ref[...] = dis * xw


def _tc2_body(a_ref, h_ref, degp_ref, b1_ref, t_ref, stats_ref):
    dis = _dis_block(degp_ref)
    t = dis * (a_ref[0] + a_ref[1] + h_ref[...]) + b1_ref[...]
    t_ref[...] = t
    stats_ref[0] = jnp.stack([jnp.sum(t, axis=0), jnp.sum(t * t, axis=0)])


def _tc3_body(t_ref, stats_ref, g_ref, be_ref, w2_ref, degp_ref, out_ref):
    stats = stats_ref[...]
    mean = jnp.sum(stats[:, 0, :], axis=0) * (1.0 / NNODE)
    ex2 = jnp.sum(stats[:, 1, :], axis=0) * (1.0 / NNODE)
    var = ex2 - mean * mean
    inv = lax.rsqrt(var + 1e-5)
    xb = (t_ref[...] - mean[None, :]) * inv[None, :] * g_ref[...] + be_ref[...]
    xb = jnp.maximum(xb, 0.0)
    s = jnp.sum(xb * xb, axis=1, keepdims=True)
    xn = xb * lax.rsqrt(jnp.maximum(s, 1e-24))
    dis = _dis_block(degp_ref)
    out_ref[...] = dis * jnp.dot(xn, w2_ref[...], precision=lax.Precision.HIGHEST)


def _tc4_body(a_ref, h_ref, degp_ref, b2_ref, out_ref):
    dis = _dis_block(degp_ref)
    out_ref[...] = dis * (a_ref[0] + a_ref[1] + h_ref[...]) + b2_ref[...]


def _tc5_body(g_ref, out_ref):
    ue = g_ref[0:B, :]
    pe = g_ref[B:2 * B, :]
    ne = g_ref[2 * B:3 * B, :]
    z = jnp.sum(ue * pe, axis=1) - jnp.sum(ue * ne, axis=1)
    ls = jnp.minimum(z, 0.0) - jnp.log1p(jnp.exp(-jnp.abs(z)))
    out_ref[...] = jnp.reshape(-jnp.sum(ls) * (1.0 / B), (1, 1))


def _row_spec(i):
    return (i, 0)


_SPEC_ND = pl.BlockSpec((BN_ROWS, D), _row_spec)
_SPEC_W = pl.BlockSpec((D, D), lambda i: (0, 0))
_SPEC_DEG = pl.BlockSpec((NC, BN_ROWS, D), lambda i: (0, i, 0))
_SPEC_AGG = pl.BlockSpec((NC, BN_ROWS, D), lambda i: (0, i, 0))
_SPEC_VEC = pl.BlockSpec((1, D), lambda i: (0, 0))


def kernel(user_emb, item_emb, W1, b1, bn_gamma, bn_beta, W2, b2,
           user_id, pos_item, neg_item, edge_index):
    x0 = jnp.concatenate([user_emb, item_emb], axis=0)
    row = edge_index[0].astype(I32)
    col = edge_index[1].astype(I32)
    pad = EPAD - row.shape[0]
    prow_flat = jnp.concatenate([row, jnp.arange(pad, dtype=I32) % NNODE])
    pcol_flat = jnp.concatenate(
        [col, NNODE + (jnp.arange(pad, dtype=I32) % (N_ACC - NNODE))])
    prow = prow_flat.reshape(NW, GCH, GCHUNK)
    pcol = pcol_flat.reshape(NW, GCH, GCHUNK)

    degp = _deg_kernel(pcol_flat.reshape(NW, CH_E, CHUNK))

    h1p = pl.pallas_call(
        _tc1_body,
        grid=(GRID_N,),
        in_specs=[_SPEC_ND, _SPEC_W, _SPEC_DEG],
        out_specs=_SPEC_ND,
        out_shape=jax.ShapeDtypeStruct((NNODE, D), F32),
    )(x0, W1, degp)

    a1 = _agg_kernel(h1p, prow, pcol)

    t, stats = pl.pallas_call(
        _tc2_body,
        grid=(GRID_N,),
        in_specs=[_SPEC_AGG, _SPEC_ND, _SPEC_DEG, _SPEC_VEC],
        out_specs=[_SPEC_ND, pl.BlockSpec((1, 2, D), lambda i: (i, 0, 0))],
        out_shape=[jax.ShapeDtypeStruct((NNODE, D), F32),
                   jax.ShapeDtypeStruct((GRID_N, 2, D), F32)],
    )(a1, h1p, degp, b1.reshape(1, D))

    h2p = pl.pallas_call(
        _tc3_body,
        grid=(GRID_N,),
        in_specs=[_SPEC_ND, pl.BlockSpec((GRID_N, 2, D), lambda i: (0, 0, 0)),
                  _SPEC_VEC, _SPEC_VEC, _SPEC_W, _SPEC_DEG],
        out_specs=_SPEC_ND,
        out_shape=jax.ShapeDtypeStruct((NNODE, D), F32),
    )(t, stats, bn_gamma.reshape(1, D), bn_beta.reshape(1, D), W2, degp)

    a2 = _agg_kernel(h2p, prow, pcol)

    xf = pl.pallas_call(
        _tc4_body,
        grid=(GRID_N,),
        in_specs=[_SPEC_AGG, _SPEC_ND, _SPEC_DEG, _SPEC_VEC],
        out_specs=_SPEC_ND,
        out_shape=jax.ShapeDtypeStruct((NNODE, D), F32),
    )(a2, h2p, degp, b2.reshape(1, D))

    idx = jnp.concatenate([
        user_id.astype(I32),
        _USER_NUM + pos_item.astype(I32),
        _USER_NUM + neg_item.astype(I32),
    ]).reshape(NW, BGK, CHUNK)
    gath = _bpr_gather(xf, idx)

    loss = pl.pallas_call(
        _tc5_body,
        in_specs=[pl.BlockSpec((3 * B, D), lambda: (0, 0))],
        out_specs=pl.BlockSpec((1, 1), lambda: (0, 0)),
        out_shape=jax.ShapeDtypeStruct((1, 1), F32),
    )(gath)

    return (loss.reshape(()), xf)


# speedup vs baseline: 1.0383x; 1.0383x over previous
import functools

import jax
import jax.numpy as jnp
from jax import lax
from jax.experimental import pallas as pl
from jax.experimental.pallas import tpu as pltpu
from jax.experimental.pallas import tpu_sc as plsc

F32 = jnp.float32
I32 = jnp.int32

_USER_NUM = 5000
_ITEM_NUM = 5000
NNODE = _USER_NUM + _ITEM_NUM
D = 128
B = 4096
NC, NS, LANES = 2, 16, 16
NW = NC * NS
CHUNK = 128
CH_E = 80
EPAD = NW * CH_E * CHUNK
N_ACC = 10240
ZROWS = N_ACC // NS
OROWS = N_ACC // NS
BGK = (3 * B) // (NW * CHUNK)

_mesh = plsc.VectorSubcoreMesh(
    core_axis_name="c", subcore_axis_name="s", num_cores=NC, num_subcores=NS)


@functools.partial(
    pl.kernel,
    out_type=jax.ShapeDtypeStruct((NC, N_ACC, D), F32),
    mesh=_mesh,
    scratch_types=[
        pltpu.VMEM((CH_E, CHUNK), I32),
        pltpu.VMEM((CHUNK, D), F32),
        pltpu.VMEM_SHARED((N_ACC, D), F32),
    ],
)
def _deg_kernel(col_hbm, out_hbm, col_v, ones_v, acc):
    cid = lax.axis_index("c")
    sid = lax.axis_index("s")
    wid = cid * NS + sid
    pltpu.sync_copy(col_hbm.at[wid], col_v)
    zero = jnp.zeros((LANES,), F32)

    def _zb(i, _):
        for k in range(D // LANES):
            ones_v[i, pl.ds(k * LANES, LANES)] = zero
        return 0

    lax.fori_loop(0, CHUNK, _zb, 0)
    base = sid * ZROWS
    for k in range(ZROWS // CHUNK):
        pltpu.sync_copy(ones_v, acc.at[pl.ds(base + k * CHUNK, CHUNK)])

    one = jnp.ones((LANES,), F32)

    def _ob(i, _):
        for k in range(D // LANES):
            ones_v[i, pl.ds(k * LANES, LANES)] = one
        return 0

    lax.fori_loop(0, CHUNK, _ob, 0)
    plsc.subcore_barrier()

    def _body(j, _):
        pltpu.sync_copy(ones_v, acc.at[col_v.at[j]], add=True)
        return 0

    lax.fori_loop(0, CH_E, _body, 0)
    plsc.subcore_barrier()
    ob = sid * OROWS
    pltpu.sync_copy(acc.at[pl.ds(ob, OROWS)], out_hbm.at[cid, pl.ds(ob, OROWS)])


@functools.partial(
    pl.kernel,
    out_type=jax.ShapeDtypeStruct((NC, N_ACC, D), F32),
    mesh=_mesh,
    scratch_types=[
        pltpu.VMEM((CH_E // 2, CHUNK), I32),
        pltpu.VMEM((CH_E // 2, CHUNK), I32),
        pltpu.VMEM((CHUNK, D), F32),
        pltpu.VMEM((CHUNK, D), F32),
        pltpu.VMEM_SHARED((N_ACC, D), F32),
        pltpu.SemaphoreType.DMA,
        pltpu.SemaphoreType.DMA,
    ],
)
def _agg_kernel(h_hbm, row_hbm, col_hbm, out_hbm, row_v, col_v, buf_a, buf_b,
                acc, sem_a, sem_b):
    cid = lax.axis_index("c")
    sid = lax.axis_index("s")
    wid = cid * NS + sid
    zero = jnp.zeros((LANES,), F32)

    def _zb(i, _):
        for k in range(D // LANES):
            buf_a[i, pl.ds(k * LANES, LANES)] = zero
        return 0

    lax.fori_loop(0, CHUNK, _zb, 0)
    base = sid * ZROWS
    for k in range(ZROWS // CHUNK):
        pltpu.sync_copy(buf_a, acc.at[pl.ds(base + k * CHUNK, CHUNK)])
    plsc.subcore_barrier()

    def _start(j, buf, sem):
        pltpu.async_copy(h_hbm.at[row_v.at[j]], buf, sem)

    def _wait(j, buf, sem):
        pltpu.make_async_copy(h_hbm.at[row_v.at[j]], buf, sem).wait()

    HALF = CH_E // 2
    NPAIR = HALF // 2
    for ph in range(2):
        pltpu.sync_copy(row_hbm.at[wid, pl.ds(ph * HALF, HALF)], row_v)
        pltpu.sync_copy(col_hbm.at[wid, pl.ds(ph * HALF, HALF)], col_v)

        def _body(jj, _):
            j0 = 2 * jj
            j1 = j0 + 1
            _start(j0, buf_a, sem_a)
            _start(j1, buf_b, sem_b)
            _wait(j0, buf_a, sem_a)
            pltpu.sync_copy(buf_a, acc.at[col_v.at[j0]], add=True)
            _wait(j1, buf_b, sem_b)
            pltpu.sync_copy(buf_b, acc.at[col_v.at[j1]], add=True)
            return 0

        lax.fori_loop(0, NPAIR, _body, 0)
    plsc.subcore_barrier()
    ob = sid * OROWS
    pltpu.sync_copy(acc.at[pl.ds(ob, OROWS)], out_hbm.at[cid, pl.ds(ob, OROWS)])


@functools.partial(
    pl.kernel,
    out_type=jax.ShapeDtypeStruct((3 * B, D), F32),
    mesh=_mesh,
    scratch_types=[
        pltpu.VMEM((BGK, CHUNK), I32),
        pltpu.VMEM((CHUNK, D), F32),
        pltpu.SemaphoreType.DMA,
    ],
)
def _bpr_gather(x_hbm, idx_hbm, out_hbm, idx_v, buf, sem):
    cid = lax.axis_index("c")
    sid = lax.axis_index("s")
    wid = cid * NS + sid
    pltpu.sync_copy(idx_hbm.at[wid], idx_v)
    for j in range(BGK):
        pltpu.async_copy(x_hbm.at[idx_v.at[j]], buf, sem).wait()
        pltpu.sync_copy(buf, out_hbm.at[pl.ds((wid * BGK + j) * CHUNK, CHUNK)])


BN_ROWS = 2000
GRID_N = NNODE // BN_ROWS


def _dis_block(degp_ref):
    deg = degp_ref[0, :, 0:1] + degp_ref[1, :, 0:1] + 1.0
    return lax.rsqrt(deg)


def _mm(a, b):
    ah = a.astype(jnp.bfloat16)
    al = (a - ah.astype(F32)).astype(jnp.bfloat16)
    bh = b.astype(jnp.bfloat16)
    bl = (b - bh.astype(F32)).astype(jnp.bfloat16)

    def dot(u, v):
        return lax.dot_general(u, v, (((1,), (0,)), ((), ())),
                               preferred_element_type=F32)

    return dot(ah, bh) + dot(ah, bl) + dot(al, bh)


def _tc1a_body(x_ref, w_ref, out_ref):
    out_ref[...] = _mm(x_ref[...], w_ref[...])


def _tc1b_body(xw_ref, degp_ref, out_ref):
    out_ref[...] = _dis_block(degp_ref) * xw_ref[...]


def _tc2_body(a_ref, h_ref, degp_ref, b1_ref, t_ref, stats_ref):
    dis = _dis_block(degp_ref)
    t = dis * (a_ref[0] + a_ref[1] + h_ref[...]) + b1_ref[...]
    t_ref[...] = t
    stats_ref[0] = jnp.stack([jnp.sum(t, axis=0), jnp.sum(t * t, axis=0)])


def _tc3_body(t_ref, stats_ref, g_ref, be_ref, w2_ref, degp_ref, out_ref):
    stats = stats_ref[...]
    mean = jnp.sum(stats[:, 0, :], axis=0) * (1.0 / NNODE)
    ex2 = jnp.sum(stats[:, 1, :], axis=0) * (1.0 / NNODE)
    var = ex2 - mean * mean
    inv = lax.rsqrt(var + 1e-5)
    xb = (t_ref[...] - mean[None, :]) * inv[None, :] * g_ref[...] + be_ref[...]
    xb = jnp.maximum(xb, 0.0)
    s = jnp.sum(xb * xb, axis=1, keepdims=True)
    xn = xb * lax.rsqrt(jnp.maximum(s, 1e-24))
    dis = _dis_block(degp_ref)
    out_ref[...] = dis * _mm(xn, w2_ref[...])


def _tc4_body(a_ref, h_ref, degp_ref, b2_ref, out_ref):
    dis = _dis_block(degp_ref)
    out_ref[...] = dis * (a_ref[0] + a_ref[1] + h_ref[...]) + b2_ref[...]


def _tc5_body(g_ref, out_ref):
    ue = g_ref[0:B, :]
    pe = g_ref[B:2 * B, :]
    ne = g_ref[2 * B:3 * B, :]
    z = jnp.sum(ue * pe, axis=1) - jnp.sum(ue * ne, axis=1)
    ls = jnp.minimum(z, 0.0) - jnp.log1p(jnp.exp(-jnp.abs(z)))
    out_ref[...] = jnp.reshape(-jnp.sum(ls) * (1.0 / B), (1, 1))


def _row_spec(i):
    return (i, 0)


_SPEC_ND = pl.BlockSpec((BN_ROWS, D), _row_spec)
_SPEC_W = pl.BlockSpec((D, D), lambda i: (0, 0))
_SPEC_DEG = pl.BlockSpec((NC, BN_ROWS, D), lambda i: (0, i, 0))
_SPEC_AGG = pl.BlockSpec((NC, BN_ROWS, D), lambda i: (0, i, 0))
_SPEC_VEC = pl.BlockSpec((1, D), lambda i: (0, 0))


def kernel(user_emb, item_emb, W1, b1, bn_gamma, bn_beta, W2, b2,
           user_id, pos_item, neg_item, edge_index):
    x0 = jnp.concatenate([user_emb, item_emb], axis=0)
    row = edge_index[0].astype(I32)
    col = edge_index[1].astype(I32)
    pad = EPAD - row.shape[0]
    prow_flat = jnp.concatenate([row, jnp.arange(pad, dtype=I32) % NNODE])
    pcol_flat = jnp.concatenate(
        [col, NNODE + (jnp.arange(pad, dtype=I32) % (N_ACC - NNODE))])
    prow = prow_flat.reshape(NW, CH_E, CHUNK)
    pcol = pcol_flat.reshape(NW, CH_E, CHUNK)

    degp = _deg_kernel(pcol)

    xw1 = pl.pallas_call(
        _tc1a_body,
        grid=(GRID_N,),
        in_specs=[_SPEC_ND, _SPEC_W],
        out_specs=_SPEC_ND,
        out_shape=jax.ShapeDtypeStruct((NNODE, D), F32),
    )(x0, W1)

    h1p = pl.pallas_call(
        _tc1b_body,
        grid=(GRID_N,),
        in_specs=[_SPEC_ND, _SPEC_DEG],
        out_specs=_SPEC_ND,
        out_shape=jax.ShapeDtypeStruct((NNODE, D), F32),
    )(xw1, degp)

    a1 = _agg_kernel(h1p, prow, pcol)

    t, stats = pl.pallas_call(
        _tc2_body,
        grid=(GRID_N,),
        in_specs=[_SPEC_AGG, _SPEC_ND, _SPEC_DEG, _SPEC_VEC],
        out_specs=[_SPEC_ND, pl.BlockSpec((1, 2, D), lambda i: (i, 0, 0))],
        out_shape=[jax.ShapeDtypeStruct((NNODE, D), F32),
                   jax.ShapeDtypeStruct((GRID_N, 2, D), F32)],
    )(a1, h1p, degp, b1.reshape(1, D))

    h2p = pl.pallas_call(
        _tc3_body,
        grid=(GRID_N,),
        in_specs=[_SPEC_ND, pl.BlockSpec((GRID_N, 2, D), lambda i: (0, 0, 0)),
                  _SPEC_VEC, _SPEC_VEC, _SPEC_W, _SPEC_DEG],
        out_specs=_SPEC_ND,
        out_shape=jax.ShapeDtypeStruct((NNODE, D), F32),
    )(t, stats, bn_gamma.reshape(1, D), bn_beta.reshape(1, D), W2, degp)

    a2 = _agg_kernel(h2p, prow, pcol)

    xf = pl.pallas_call(
        _tc4_body,
        grid=(GRID_N,),
        in_specs=[_SPEC_AGG, _SPEC_ND, _SPEC_DEG, _SPEC_VEC],
        out_specs=_SPEC_ND,
        out_shape=jax.ShapeDtypeStruct((NNODE, D), F32),
    )(a2, h2p, degp, b2.reshape(1, D))

    idx = jnp.concatenate([
        user_id.astype(I32),
        _USER_NUM + pos_item.astype(I32),
        _USER_NUM + neg_item.astype(I32),
    ]).reshape(NW, BGK, CHUNK)
    gath = _bpr_gather(xf, idx)

    loss = pl.pallas_call(
        _tc5_body,
        in_specs=[pl.BlockSpec((3 * B, D), lambda: (0, 0))],
        out_specs=pl.BlockSpec((1, 1), lambda: (0, 0)),
        out_shape=jax.ShapeDtypeStruct((1, 1), F32),
    )(gath)

    return (loss.reshape(()), xf)


# speedup vs baseline: 1.0398x; 1.0014x over previous
import functools

import jax
import jax.numpy as jnp
from jax import lax
from jax.experimental import pallas as pl
from jax.experimental.pallas import tpu as pltpu
from jax.experimental.pallas import tpu_sc as plsc

F32 = jnp.float32
I32 = jnp.int32

_USER_NUM = 5000
_ITEM_NUM = 5000
NNODE = _USER_NUM + _ITEM_NUM
D = 128
B = 4096
NC, NS, LANES = 2, 16, 16
NW = NC * NS
CHUNK = 128
CH_E = 80
EPAD = NW * CH_E * CHUNK
N_ACC = 10240
ZROWS = N_ACC // NS
OROWS = N_ACC // NS
BGK = (3 * B) // (NW * CHUNK)

_mesh = plsc.VectorSubcoreMesh(
    core_axis_name="c", subcore_axis_name="s", num_cores=NC, num_subcores=NS)


@functools.partial(
    pl.kernel,
    out_type=jax.ShapeDtypeStruct((NC, N_ACC, D), F32),
    mesh=_mesh,
    scratch_types=[
        pltpu.VMEM((CH_E, CHUNK), I32),
        pltpu.VMEM((CHUNK, D), F32),
        pltpu.VMEM_SHARED((N_ACC, D), F32),
    ],
)
def _deg_kernel(col_hbm, out_hbm, col_v, ones_v, acc):
    cid = lax.axis_index("c")
    sid = lax.axis_index("s")
    wid = cid * NS + sid
    pltpu.sync_copy(col_hbm.at[wid], col_v)
    zero = jnp.zeros((LANES,), F32)

    def _zb(i, _):
        for k in range(D // LANES):
            ones_v[i, pl.ds(k * LANES, LANES)] = zero
        return 0

    lax.fori_loop(0, CHUNK, _zb, 0)
    base = sid * ZROWS
    for k in range(ZROWS // CHUNK):
        pltpu.sync_copy(ones_v, acc.at[pl.ds(base + k * CHUNK, CHUNK)])

    one = jnp.ones((LANES,), F32)

    def _ob(i, _):
        for k in range(D // LANES):
            ones_v[i, pl.ds(k * LANES, LANES)] = one
        return 0

    lax.fori_loop(0, CHUNK, _ob, 0)
    plsc.subcore_barrier()

    def _body(j, _):
        pltpu.sync_copy(ones_v, acc.at[col_v.at[j]], add=True)
        return 0

    lax.fori_loop(0, CH_E, _body, 0)
    plsc.subcore_barrier()
    ob = sid * OROWS
    pltpu.sync_copy(acc.at[pl.ds(ob, OROWS)], out_hbm.at[cid, pl.ds(ob, OROWS)])


@functools.partial(
    pl.kernel,
    out_type=jax.ShapeDtypeStruct((NC, N_ACC, D), F32),
    mesh=_mesh,
    scratch_types=[
        pltpu.VMEM((CH_E // 2, CHUNK), I32),
        pltpu.VMEM((CH_E // 2, CHUNK), I32),
        pltpu.VMEM((CHUNK, D), F32),
        pltpu.VMEM((CHUNK, D), F32),
        pltpu.VMEM_SHARED((N_ACC, D), F32),
        pltpu.SemaphoreType.DMA,
        pltpu.SemaphoreType.DMA,
    ],
)
def _agg_kernel(h_hbm, row_hbm, col_hbm, out_hbm, row_v, col_v, buf_a, buf_b,
                acc, sem_a, sem_b):
    cid = lax.axis_index("c")
    sid = lax.axis_index("s")
    wid = cid * NS + sid
    zero = jnp.zeros((LANES,), F32)

    def _zb(i, _):
        for k in range(D // LANES):
            buf_a[i, pl.ds(k * LANES, LANES)] = zero
        return 0

    lax.fori_loop(0, CHUNK, _zb, 0)
    base = sid * ZROWS
    for k in range(ZROWS // CHUNK):
        pltpu.sync_copy(buf_a, acc.at[pl.ds(base + k * CHUNK, CHUNK)])
    plsc.subcore_barrier()

    def _start(j, buf, sem):
        pltpu.async_copy(h_hbm.at[row_v.at[j]], buf, sem)

    def _wait(j, buf, sem):
        pltpu.make_async_copy(h_hbm.at[row_v.at[j]], buf, sem).wait()

    HALF = CH_E // 2
    NPAIR = HALF // 2
    for ph in range(2):
        pltpu.sync_copy(row_hbm.at[wid, pl.ds(ph * HALF, HALF)], row_v)
        pltpu.sync_copy(col_hbm.at[wid, pl.ds(ph * HALF, HALF)], col_v)

        def _body(jj, _):
            j0 = 2 * jj
            j1 = j0 + 1
            _start(j0, buf_a, sem_a)
            _start(j1, buf_b, sem_b)
            _wait(j0, buf_a, sem_a)
            pltpu.sync_copy(buf_a, acc.at[col_v.at[j0]], add=True)
            _wait(j1, buf_b, sem_b)
            pltpu.sync_copy(buf_b, acc.at[col_v.at[j1]], add=True)
            return 0

        lax.fori_loop(0, NPAIR, _body, 0)
    plsc.subcore_barrier()
    ob = sid * OROWS
    pltpu.sync_copy(acc.at[pl.ds(ob, OROWS)], out_hbm.at[cid, pl.ds(ob, OROWS)])


@functools.partial(
    pl.kernel,
    out_type=jax.ShapeDtypeStruct((3 * B, D), F32),
    mesh=_mesh,
    scratch_types=[
        pltpu.VMEM((BGK, CHUNK), I32),
        pltpu.VMEM((CHUNK, D), F32),
        pltpu.SemaphoreType.DMA,
    ],
)
def _bpr_gather(x_hbm, idx_hbm, out_hbm, idx_v, buf, sem):
    cid = lax.axis_index("c")
    sid = lax.axis_index("s")
    wid = cid * NS + sid
    pltpu.sync_copy(idx_hbm.at[wid], idx_v)
    for j in range(BGK):
        pltpu.async_copy(x_hbm.at[idx_v.at[j]], buf, sem).wait()
        pltpu.sync_copy(buf, out_hbm.at[pl.ds((wid * BGK + j) * CHUNK, CHUNK)])


BN_ROWS = 2000
GRID_N = NNODE // BN_ROWS


def _dis_block(degp_ref):
    deg = degp_ref[0, :, 0:1] + degp_ref[1, :, 0:1] + 1.0
    return lax.rsqrt(deg)


def _mm(a, b):
    ah = a.astype(jnp.bfloat16)
    al = (a - ah.astype(F32)).astype(jnp.bfloat16)
    bh = b.astype(jnp.bfloat16)
    bl = (b - bh.astype(F32)).astype(jnp.bfloat16)

    def dot(u, v):
        return lax.dot_general(u, v, (((1,), (0,)), ((), ())),
                               preferred_element_type=F32)

    return dot(ah, bh) + dot(ah, bl) + dot(al, bh)


def _tc1a_body(x_ref, w_ref, out_ref):
    out_ref[...] = _mm(x_ref[...], w_ref[...])


def _tc1b_body(xw_ref, degp_ref, out_ref, dis8_ref):
    dis = _dis_block(degp_ref)
    out_ref[...] = dis * xw_ref[...]
    dis8_ref[...] = jnp.broadcast_to(dis, (dis.shape[0], 8))


def _tc2_body(a_ref, h_ref, dis8_ref, b1_ref, t_ref, stats_ref):
    dis = dis8_ref[:, 0:1]
    t = dis * (a_ref[0] + a_ref[1] + h_ref[...]) + b1_ref[...]
    t_ref[...] = t
    stats_ref[0] = jnp.stack([jnp.sum(t, axis=0), jnp.sum(t * t, axis=0)])


def _tc3_body(t_ref, stats_ref, g_ref, be_ref, w2_ref, dis8_ref, out_ref):
    stats = stats_ref[...]
    mean = jnp.sum(stats[:, 0, :], axis=0) * (1.0 / NNODE)
    ex2 = jnp.sum(stats[:, 1, :], axis=0) * (1.0 / NNODE)
    var = ex2 - mean * mean
    inv = lax.rsqrt(var + 1e-5)
    xb = (t_ref[...] - mean[None, :]) * inv[None, :] * g_ref[...] + be_ref[...]
    xb = jnp.maximum(xb, 0.0)
    s = jnp.sum(xb * xb, axis=1, keepdims=True)
    xn = xb * lax.rsqrt(jnp.maximum(s, 1e-24))
    dis = dis8_ref[:, 0:1]
    out_ref[...] = dis * _mm(xn, w2_ref[...])


def _tc4_body(a_ref, h_ref, dis8_ref, b2_ref, out_ref):
    dis = dis8_ref[:, 0:1]
    out_ref[...] = dis * (a_ref[0] + a_ref[1] + h_ref[...]) + b2_ref[...]


def _tc5_body(g_ref, out_ref):
    ue = g_ref[0:B, :]
    pe = g_ref[B:2 * B, :]
    ne = g_ref[2 * B:3 * B, :]
    z = jnp.sum(ue * pe, axis=1) - jnp.sum(ue * ne, axis=1)
    ls = jnp.minimum(z, 0.0) - jnp.log1p(jnp.exp(-jnp.abs(z)))
    out_ref[...] = jnp.reshape(-jnp.sum(ls) * (1.0 / B), (1, 1))


def _row_spec(i):
    return (i, 0)


_SPEC_ND = pl.BlockSpec((BN_ROWS, D), _row_spec)
_SPEC_W = pl.BlockSpec((D, D), lambda i: (0, 0))
_SPEC_DEG = pl.BlockSpec((NC, BN_ROWS, D), lambda i: (0, i, 0))
_SPEC_AGG = pl.BlockSpec((NC, BN_ROWS, D), lambda i: (0, i, 0))
_SPEC_VEC = pl.BlockSpec((1, D), lambda i: (0, 0))


def kernel(user_emb, item_emb, W1, b1, bn_gamma, bn_beta, W2, b2,
           user_id, pos_item, neg_item, edge_index):
    x0 = jnp.concatenate([user_emb, item_emb], axis=0)
    row = edge_index[0].astype(I32)
    col = edge_index[1].astype(I32)
    pad = EPAD - row.shape[0]
    prow_flat = jnp.concatenate([row, jnp.arange(pad, dtype=I32) % NNODE])
    pcol_flat = jnp.concatenate(
        [col, NNODE + (jnp.arange(pad, dtype=I32) % (N_ACC - NNODE))])
    prow = prow_flat.reshape(NW, CH_E, CHUNK)
    pcol = pcol_flat.reshape(NW, CH_E, CHUNK)

    degp = _deg_kernel(pcol)

    xw1 = pl.pallas_call(
        _tc1a_body,
        grid=(GRID_N,),
        in_specs=[_SPEC_ND, _SPEC_W],
        out_specs=_SPEC_ND,
        out_shape=jax.ShapeDtypeStruct((NNODE, D), F32),
    )(x0, W1)

    h1p, dis8 = pl.pallas_call(
        _tc1b_body,
        grid=(GRID_N,),
        in_specs=[_SPEC_ND, _SPEC_DEG],
        out_specs=[_SPEC_ND, pl.BlockSpec((BN_ROWS, 8), _row_spec)],
        out_shape=[jax.ShapeDtypeStruct((NNODE, D), F32),
                   jax.ShapeDtypeStruct((NNODE, 8), F32)],
    )(xw1, degp)

    a1 = _agg_kernel(h1p, prow, pcol)

    t, stats = pl.pallas_call(
        _tc2_body,
        grid=(GRID_N,),
        in_specs=[_SPEC_AGG, _SPEC_ND, pl.BlockSpec((BN_ROWS, 8), _row_spec),
                  _SPEC_VEC],
        out_specs=[_SPEC_ND, pl.BlockSpec((1, 2, D), lambda i: (i, 0, 0))],
        out_shape=[jax.ShapeDtypeStruct((NNODE, D), F32),
                   jax.ShapeDtypeStruct((GRID_N, 2, D), F32)],
    )(a1, h1p, dis8, b1.reshape(1, D))

    h2p = pl.pallas_call(
        _tc3_body,
        grid=(GRID_N,),
        in_specs=[_SPEC_ND, pl.BlockSpec((GRID_N, 2, D), lambda i: (0, 0, 0)),
                  _SPEC_VEC, _SPEC_VEC, _SPEC_W,
                  pl.BlockSpec((BN_ROWS, 8), _row_spec)],
        out_specs=_SPEC_ND,
        out_shape=jax.ShapeDtypeStruct((NNODE, D), F32),
    )(t, stats, bn_gamma.reshape(1, D), bn_beta.reshape(1, D), W2, dis8)

    a2 = _agg_kernel(h2p, prow, pcol)

    xf = pl.pallas_call(
        _tc4_body,
        grid=(GRID_N,),
        in_specs=[_SPEC_AGG, _SPEC_ND, pl.BlockSpec((BN_ROWS, 8), _row_spec),
                  _SPEC_VEC],
        out_specs=_SPEC_ND,
        out_shape=jax.ShapeDtypeStruct((NNODE, D), F32),
    )(a2, h2p, dis8, b2.reshape(1, D))

    idx = jnp.concatenate([
        user_id.astype(I32),
        _USER_NUM + pos_item.astype(I32),
        _USER_NUM + neg_item.astype(I32),
    ]).reshape(NW, BGK, CHUNK)
    gath = _bpr_gather(xf, idx)

    loss = pl.pallas_call(
        _tc5_body,
        in_specs=[pl.BlockSpec((3 * B, D), lambda: (0, 0))],
        out_specs=pl.BlockSpec((1, 1), lambda: (0, 0)),
        out_shape=jax.ShapeDtypeStruct((1, 1), F32),
    )(gath)

    return (loss.reshape(()), xf)
